# baseline (speedup 1.0000x reference)
import functools

import jax
import jax.numpy as jnp
from jax import lax
from jax.experimental import pallas as pl
from jax.experimental.pallas import tpu as pltpu
from jax.experimental.pallas import tpu_sc as plsc

N = 10000
E = 320000
F = 128
D = 128
G = 256

NC = 2
NS = 16
NW = NC * NS

C = 40
CHUNKS = E // C
CPW = CHUNKS // NW
ST = 5
SB = CPW // ST
NB = 6
NP = 10240
RPT = NP // NS


def _seg_sum_sc(x, ei, zrows):
    mesh = plsc.VectorSubcoreMesh(core_axis_name="c", subcore_axis_name="s")

    @functools.partial(
        pl.kernel,
        mesh=mesh,
        out_type=jax.ShapeDtypeStruct((NC, NP, D), jnp.float32),
        scratch_types=[
            pltpu.VMEM((SB, C), jnp.int32),
            pltpu.VMEM((SB, C), jnp.int32),
            [pltpu.VMEM((C, D), jnp.float32) for _ in range(NB)],
            [pltpu.SemaphoreType.DMA for _ in range(NB)],
            [pltpu.SemaphoreType.DMA for _ in range(NB)],
            pltpu.VMEM_SHARED((NP, D), jnp.float32),
        ],
    )
    def k(x_hbm, ei_hbm, z_hbm, out_hbm,
          srcb, dstb, rows, gsems, ssems, acc):
        c = lax.axis_index("c")
        s = lax.axis_index("s")
        w = s * NC + c

        pltpu.sync_copy(z_hbm, acc.at[pl.ds(s * RPT, RPT)])
        plsc.subcore_barrier()

        def stage(st, _):
            pltpu.sync_copy(ei_hbm.at[0, w, st], srcb)
            pltpu.sync_copy(ei_hbm.at[1, w, st], dstb)
            for j in range(NB):
                pltpu.async_copy(x_hbm.at[srcb.at[j]], rows[j], gsems[j])

            def body(i, _):
                g0 = i * NB
                for j in range(NB):
                    g = g0 + j
                    pltpu.make_async_copy(
                        x_hbm.at[srcb.at[g]], rows[j], gsems[j]).wait()
                    pltpu.async_copy(rows[j], acc.at[dstb.at[g]], ssems[j],
                                     add=True)
                for j in range(NB):
                    g = g0 + j
                    pltpu.make_async_copy(
                        rows[j], acc.at[dstb.at[g]], ssems[j]).wait()

                    @pl.when(g + NB < SB)
                    def _():
                        pltpu.async_copy(x_hbm.at[srcb.at[g + NB]], rows[j],
                                         gsems[j])
                return 0

            lax.fori_loop(0, SB // NB, body, 0)
            for j in range(SB % NB):
                gt = (SB // NB) * NB + j
                pltpu.make_async_copy(
                    x_hbm.at[srcb.at[gt]], rows[j], gsems[j]).wait()
                pltpu.sync_copy(rows[j], acc.at[dstb.at[gt]], add=True)
            return 0

        lax.fori_loop(0, ST, stage, 0)
        plsc.subcore_barrier()

        pltpu.sync_copy(acc.at[pl.ds(s * RPT, RPT)],
                        out_hbm.at[c, pl.ds(s * RPT, RPT)])

    return k(x, ei, zrows)


def _tc_layer_body(x_ref, p_ref, w1_ref, b1_ref, w2_ref, b2_ref,
                   g_ref, be_ref, bt_ref, h_ref, pool_ref):
    h0 = x_ref[...] + p_ref[0, :N] + p_ref[1, :N]
    t = jnp.maximum(
        jnp.dot(h0, w1_ref[...], preferred_element_type=jnp.float32)
        + b1_ref[...], 0.0)
    u = (jnp.dot(t, w2_ref[...], preferred_element_type=jnp.float32)
         + b2_ref[...])
    pre = jnp.maximum(u, 0.0)
    mean = jnp.sum(pre, axis=0, keepdims=True) / N
    var = jnp.sum(pre * pre, axis=0, keepdims=True) / N - mean * mean
    h = (pre - mean) * lax.rsqrt(var + 1e-5) * g_ref[...] + be_ref[...]
    h_ref[...] = h
    onehot = (bt_ref[...] == lax.broadcasted_iota(jnp.int32, (G, 1), 0)
              ).astype(jnp.float32)
    pool_ref[...] = jnp.dot(onehot, h, preferred_element_type=jnp.float32)


def _tc_layer(x, parts, W1, b1, W2, b2, gamma, beta, batch2d):
    return pl.pallas_call(
        _tc_layer_body,
        out_shape=(
            jax.ShapeDtypeStruct((N, D), jnp.float32),
            jax.ShapeDtypeStruct((G, D), jnp.float32),
        ),
    )(x, parts, W1, b1.reshape(1, D), W2, b2.reshape(1, D),
      gamma.reshape(1, D), beta.reshape(1, D), batch2d)


def _tc_final_body(x_ref, p_ref, w1_ref, b1_ref, w2_ref, b2_ref,
                   g_ref, be_ref, bt_ref, h0_ref, h1_ref, q0_ref, q1_ref,
                   hcat_ref, pcat_ref):
    h0 = x_ref[...] + p_ref[0, :N] + p_ref[1, :N]
    t = jnp.maximum(
        jnp.dot(h0, w1_ref[...], preferred_element_type=jnp.float32)
        + b1_ref[...], 0.0)
    u = (jnp.dot(t, w2_ref[...], preferred_element_type=jnp.float32)
         + b2_ref[...])
    pre = jnp.maximum(u, 0.0)
    mean = jnp.sum(pre, axis=0, keepdims=True) / N
    var = jnp.sum(pre * pre, axis=0, keepdims=True) / N - mean * mean
    h = (pre - mean) * lax.rsqrt(var + 1e-5) * g_ref[...] + be_ref[...]
    onehot = (bt_ref[...] == lax.broadcasted_iota(jnp.int32, (G, 1), 0)
              ).astype(jnp.float32)
    pool = jnp.dot(onehot, h, preferred_element_type=jnp.float32)
    hcat_ref[:, :D] = h0_ref[...]
    hcat_ref[:, D:2 * D] = h1_ref[...]
    hcat_ref[:, 2 * D:] = h
    pcat_ref[:, :D] = q0_ref[...]
    pcat_ref[:, D:2 * D] = q1_ref[...]
    pcat_ref[:, 2 * D:] = pool


def _tc_final(x, parts, W1, b1, W2, b2, gamma, beta, batch2d,
              h0, h1, q0, q1):
    return pl.pallas_call(
        _tc_final_body,
        out_shape=(
            jax.ShapeDtypeStruct((N, 3 * D), jnp.float32),
            jax.ShapeDtypeStruct((G, 3 * D), jnp.float32),
        ),
    )(x, parts, W1, b1.reshape(1, D), W2, b2.reshape(1, D),
      gamma.reshape(1, D), beta.reshape(1, D), batch2d, h0, h1, q0, q1)


def kernel(x, edge_index, batch,
           W1_0, b1_0, W2_0, b2_0, gamma_0, beta_0,
           W1_1, b1_1, W2_1, b2_1, gamma_1, beta_1,
           W1_2, b1_2, W2_2, b2_2, gamma_2, beta_2):
    ei = edge_index.reshape(2, NW, ST, SB, C)
    zrows = jnp.zeros((RPT, D), dtype=jnp.float32)
    batch2d = batch.reshape(1, N)

    parts = _seg_sum_sc(x, ei, zrows)
    h0, q0 = _tc_layer(x, parts, W1_0, b1_0, W2_0, b2_0, gamma_0, beta_0,
                       batch2d)
    parts = _seg_sum_sc(h0, ei, zrows)
    h1, q1 = _tc_layer(h0, parts, W1_1, b1_1, W2_1, b2_1, gamma_1, beta_1,
                       batch2d)
    parts = _seg_sum_sc(h1, ei, zrows)
    hcat, pcat = _tc_final(h1, parts, W1_2, b1_2, W2_2, b2_2, gamma_2,
                           beta_2, batch2d, h0, h1, q0, q1)
    return (pcat, hcat)


# speedup vs baseline: 1.0243x; 1.0243x over previous
import functools

import jax
import jax.numpy as jnp
from jax import lax
from jax.experimental import pallas as pl
from jax.experimental.pallas import tpu as pltpu
from jax.experimental.pallas import tpu_sc as plsc

N = 10000
E = 320000
F = 128
D = 128
G = 256

NC = 2
NS = 16
NW = NC * NS

C = 80
CHUNKS = E // C
CPW = CHUNKS // NW
ST = 5
SB = CPW // ST
NB = 4
NP = 10240
RPT = NP // NS


def _seg_sum_sc(x, ei, zrows):
    mesh = plsc.VectorSubcoreMesh(core_axis_name="c", subcore_axis_name="s")

    @functools.partial(
        pl.kernel,
        mesh=mesh,
        out_type=jax.ShapeDtypeStruct((NC, NP, D), jnp.float32),
        scratch_types=[
            pltpu.VMEM((SB, C), jnp.int32),
            pltpu.VMEM((SB, C), jnp.int32),
            [pltpu.VMEM((C, D), jnp.float32) for _ in range(NB)],
            [pltpu.SemaphoreType.DMA for _ in range(NB)],
            [pltpu.SemaphoreType.DMA for _ in range(NB)],
            pltpu.VMEM_SHARED((NP, D), jnp.float32),
        ],
    )
    def k(x_hbm, ei_hbm, z_hbm, out_hbm,
          srcb, dstb, rows, gsems, ssems, acc):
        c = lax.axis_index("c")
        s = lax.axis_index("s")
        w = s * NC + c

        pltpu.sync_copy(ei_hbm.at[0, w, 0], srcb)
        pltpu.sync_copy(ei_hbm.at[1, w, 0], dstb)
        for j in range(NB):
            pltpu.async_copy(x_hbm.at[srcb.at[j]], rows[j], gsems[j])
        pltpu.sync_copy(z_hbm, acc.at[pl.ds(s * RPT, RPT)])
        plsc.subcore_barrier()

        def stage(st, _):
            def body(i, _):
                g0 = i * NB
                for j in range(NB):
                    g = g0 + j
                    pltpu.make_async_copy(
                        x_hbm.at[srcb.at[g]], rows[j], gsems[j]).wait()
                    pltpu.async_copy(rows[j], acc.at[dstb.at[g]], ssems[j],
                                     add=True)
                for j in range(NB):
                    g = g0 + j
                    pltpu.make_async_copy(
                        rows[j], acc.at[dstb.at[g]], ssems[j]).wait()

                    @pl.when(g + NB < SB)
                    def _():
                        pltpu.async_copy(x_hbm.at[srcb.at[g + NB]], rows[j],
                                         gsems[j])
                return 0

            lax.fori_loop(0, SB // NB, body, 0)
            for j in range(SB % NB):
                gt = (SB // NB) * NB + j
                pltpu.make_async_copy(
                    x_hbm.at[srcb.at[gt]], rows[j], gsems[j]).wait()
                pltpu.sync_copy(rows[j], acc.at[dstb.at[gt]], add=True)

            @pl.when(st < ST - 1)
            def _():
                pltpu.sync_copy(ei_hbm.at[0, w, st + 1], srcb)
                pltpu.sync_copy(ei_hbm.at[1, w, st + 1], dstb)
                for j in range(NB):
                    pltpu.async_copy(x_hbm.at[srcb.at[j]], rows[j], gsems[j])
            return 0

        lax.fori_loop(0, ST, stage, 0)
        plsc.subcore_barrier()

        pltpu.sync_copy(acc.at[pl.ds(s * RPT, RPT)],
                        out_hbm.at[c, pl.ds(s * RPT, RPT)])

    return k(x, ei, zrows)


def _tc_layer_body(x_ref, p_ref, w1_ref, b1_ref, w2_ref, b2_ref,
                   g_ref, be_ref, bt_ref, h_ref, pool_ref):
    h0 = x_ref[...] + p_ref[0, :N] + p_ref[1, :N]
    t = jnp.maximum(
        jnp.dot(h0, w1_ref[...], preferred_element_type=jnp.float32)
        + b1_ref[...], 0.0)
    u = (jnp.dot(t, w2_ref[...], preferred_element_type=jnp.float32)
         + b2_ref[...])
    pre = jnp.maximum(u, 0.0)
    mean = jnp.sum(pre, axis=0, keepdims=True) / N
    var = jnp.sum(pre * pre, axis=0, keepdims=True) / N - mean * mean
    h = (pre - mean) * lax.rsqrt(var + 1e-5) * g_ref[...] + be_ref[...]
    h_ref[...] = h
    onehot = (bt_ref[...] == lax.broadcasted_iota(jnp.int32, (G, 1), 0)
              ).astype(jnp.float32)
    pool_ref[...] = jnp.dot(onehot, h, preferred_element_type=jnp.float32)


def _tc_layer(x, parts, W1, b1, W2, b2, gamma, beta, batch2d):
    return pl.pallas_call(
        _tc_layer_body,
        out_shape=(
            jax.ShapeDtypeStruct((N, D), jnp.float32),
            jax.ShapeDtypeStruct((G, D), jnp.float32),
        ),
    )(x, parts, W1, b1.reshape(1, D), W2, b2.reshape(1, D),
      gamma.reshape(1, D), beta.reshape(1, D), batch2d)


def _tc_final_body(x_ref, p_ref, w1_ref, b1_ref, w2_ref, b2_ref,
                   g_ref, be_ref, bt_ref, h0_ref, h1_ref, q0_ref, q1_ref,
                   hcat_ref, pcat_ref):
    h0 = x_ref[...] + p_ref[0, :N] + p_ref[1, :N]
    t = jnp.maximum(
        jnp.dot(h0, w1_ref[...], preferred_element_type=jnp.float32)
        + b1_ref[...], 0.0)
    u = (jnp.dot(t, w2_ref[...], preferred_element_type=jnp.float32)
         + b2_ref[...])
    pre = jnp.maximum(u, 0.0)
    mean = jnp.sum(pre, axis=0, keepdims=True) / N
    var = jnp.sum(pre * pre, axis=0, keepdims=True) / N - mean * mean
    h = (pre - mean) * lax.rsqrt(var + 1e-5) * g_ref[...] + be_ref[...]
    onehot = (bt_ref[...] == lax.broadcasted_iota(jnp.int32, (G, 1), 0)
              ).astype(jnp.float32)
    pool = jnp.dot(onehot, h, preferred_element_type=jnp.float32)
    hcat_ref[:, :D] = h0_ref[...]
    hcat_ref[:, D:2 * D] = h1_ref[...]
    hcat_ref[:, 2 * D:] = h
    pcat_ref[:, :D] = q0_ref[...]
    pcat_ref[:, D:2 * D] = q1_ref[...]
    pcat_ref[:, 2 * D:] = pool


def _tc_final(x, parts, W1, b1, W2, b2, gamma, beta, batch2d,
              h0, h1, q0, q1):
    return pl.pallas_call(
        _tc_final_body,
        out_shape=(
            jax.ShapeDtypeStruct((N, 3 * D), jnp.float32),
            jax.ShapeDtypeStruct((G, 3 * D), jnp.float32),
        ),
    )(x, parts, W1, b1.reshape(1, D), W2, b2.reshape(1, D),
      gamma.reshape(1, D), beta.reshape(1, D), batch2d, h0, h1, q0, q1)


def kernel(x, edge_index, batch,
           W1_0, b1_0, W2_0, b2_0, gamma_0, beta_0,
           W1_1, b1_1, W2_1, b2_1, gamma_1, beta_1,
           W1_2, b1_2, W2_2, b2_2, gamma_2, beta_2):
    ei = edge_index.reshape(2, NW, ST, SB, C)
    zrows = jnp.zeros((RPT, D), dtype=jnp.float32)
    batch2d = batch.reshape(1, N)

    parts = _seg_sum_sc(x, ei, zrows)
    h0, q0 = _tc_layer(x, parts, W1_0, b1_0, W2_0, b2_0, gamma_0, beta_0,
                       batch2d)
    parts = _seg_sum_sc(h0, ei, zrows)
    h1, q1 = _tc_layer(h0, parts, W1_1, b1_1, W2_1, b2_1, gamma_1, beta_1,
                       batch2d)
    parts = _seg_sum_sc(h1, ei, zrows)
    hcat, pcat = _tc_final(h1, parts, W1_2, b1_2, W2_2, b2_2, gamma_2,
                           beta_2, batch2d, h0, h1, q0, q1)
    return (pcat, hcat)
